# Initial kernel scaffold; baseline (speedup 1.0000x reference)
import jax
import jax.numpy as jnp
from jax.experimental import pallas as pl


def kernel(x, edge_index, edge_attr, edge_sbf, node_rbf, batch, is_cleave, num_graphs, params):
    raise NotImplementedError("write your pallas kernel here")


# speedup vs baseline: 17.3118x; 17.3118x over previous
import functools

import jax
import jax.numpy as jnp
from jax import lax
from jax.experimental import pallas as pl
from jax.experimental.pallas import tpu as pltpu
from jax.experimental.pallas import tpu_sc as plsc

N_ = 10000
E_ = 160000
F_ = 128
H_ = 8
CH_ = 16
G_ = 128
EPS = 1e-8

BE = 2000
BN = 2000

_NW = 32
_SC_CHUNK = 40
_SC_ITERS = E_ // (_SC_CHUNK * _NW)
_ZROWS = 80
_NZCH = N_ // _ZROWS
_ZITER = -(-_NZCH // 16)


def _silu(t):
    return t * jax.nn.sigmoid(t)


def _dot(a, b):
    return jnp.dot(a, b, precision=lax.Precision.HIGHEST)


def _dotw(a, b):
    return jnp.dot(a, b, precision=lax.Precision.DEFAULT)


def _dotc0(a, b):
    return lax.dot_general(a, b, (((0,), (0,)), ((), ())),
                           precision=lax.Precision.HIGHEST)


def _head_mats():
    hsel = (lax.broadcasted_iota(jnp.int32, (F_, H_), 0) // CH_
            == lax.broadcasted_iota(jnp.int32, (F_, H_), 1)).astype(jnp.float32)
    hexp = (lax.broadcasted_iota(jnp.int32, (H_, F_), 1) // CH_
            == lax.broadcasted_iota(jnp.int32, (H_, F_), 0)).astype(jnp.float32)
    return hsel, hexp


def _onehot_graph(batch_col):
    return (batch_col == lax.broadcasted_iota(jnp.int32, (N_, G_), 1)
            ).astype(jnp.float32)


def _k1_body(ea_r, sbf_r, rbf_r, W0r, b0r, W1r, b1r,
             Wekr, bekr, Wevr, bevr, Wgr, bgr, Wsr, bsr, *outs):
    a = _dotw(_silu(_dotw(ea_r[...], W0r[...]) + b0r[...]), W1r[...]) + b1r[...]
    sbf = sbf_r[...]
    rbf = rbf_r[...]
    Wek, bek = Wekr[...], bekr[...]
    Wev, bev = Wevr[...], bevr[...]
    Wg, bg = Wgr[...], bgr[...]
    Ws, bs = Wsr[...], bsr[...]
    for i in range(3):
        outs[i][...] = _dotw(a, Wek[i]) + bek[i][None, :]
        outs[3 + i][...] = _dotw(a, Wev[i]) + bev[i][None, :]
        outs[6 + i][...] = _dotw(rbf, Wg[i]) + bg[i][None, :]
        outs[9 + i][...] = _dotw(sbf, Ws[i]) + bs[i][None, :]


def _k2_body(x_r, Wq, bq, Wk, bk, Wv, bv, q_o, k_o, v_o):
    xv = x_r[...]
    q_o[...] = _dotw(xv, Wq[...]) + bq[...]
    k_o[...] = _dotw(xv, Wk[...]) + bk[...]
    v_o[...] = _dotw(xv, Wv[...]) + bv[...]


def _k4_body(qg_r, kg_r, vg_r, ek_r, ev_r, g_r, s_r, a0_o, a1_o):
    hsel, hexp = _head_mats()
    qg = qg_r[...]
    kk = kg_r[...] + ek_r[...]
    logits = _dot(qg * kk, hsel) * 0.25 + s_r[...]
    ehf = _dot(jnp.exp(logits), hexp)
    m = ehf * (vg_r[...] + ev_r[...]) * g_r[...]
    a0_o[...] = jnp.concatenate([m[:, :64], ehf[:, :64]], axis=1)
    a1_o[...] = jnp.concatenate([m[:, 64:], ehf[:, 64:]], axis=1)


def _onehot_block(batch_col):
    return (batch_col == lax.broadcasted_iota(jnp.int32, (BN, G_), 1)
            ).astype(jnp.float32)


def _k6a_body(hs_r, batch_r, h_o, S1_o, S2_o, cnt_o):
    i = pl.program_id(0)
    hs = hs_r[...]
    h = jnp.concatenate(
        [hs[0, :, :64] / (hs[0, :, 64:] + 1e-16),
         hs[1, :, :64] / (hs[1, :, 64:] + 1e-16)], axis=1)
    h_o[...] = h
    oh = _onehot_block(batch_r[...])
    ones = jnp.full((BN, 1), 1.0, jnp.float32)

    @pl.when(i == 0)
    def _():
        S1_o[...] = jnp.zeros_like(S1_o)
        S2_o[...] = jnp.zeros_like(S2_o)
        cnt_o[...] = jnp.zeros_like(cnt_o)

    S1_o[...] += _dotc0(oh, h)
    S2_o[...] += _dotc0(oh, h * h)
    cnt_o[...] += _dotc0(oh, ones)


def _k6b_body(h_r, batch_r, S1_r, S2_r, cnt_r, res0_r,
              Wb1, bb1, Wb2, bb2, Wd, bd,
              Wa1, ba1, Wa2, ba2, Wa3, ba3, Wa4, ba4, out_o):
    n = cnt_r[...] * jnp.float32(F_) + 1e-12
    mean = jnp.sum(S1_r[...], axis=1, keepdims=True) / n
    var = jnp.sum(S2_r[...], axis=1, keepdims=True) / n - mean * mean
    sd = jnp.sqrt(var + EPS) + EPS
    oh = _onehot_block(batch_r[...])
    h = (h_r[...] - _dot(oh, mean)) / _dot(oh, sd)
    h = h + _silu(_dotw(_silu(_dotw(h, Wb1[...]) + bb1[...]), Wb2[...]) + bb2[...])
    h = _silu(_dotw(h, Wd[...]) + bd[...])
    h = h + res0_r[...]
    h = h + _silu(_dotw(_silu(_dotw(h, Wa1[...]) + ba1[...]), Wa2[...]) + ba2[...])
    h = h + _silu(_dotw(_silu(_dotw(h, Wa3[...]) + ba3[...]), Wa4[...]) + ba4[...])
    out_o[...] = h


def _k7_body(out_r, batch_r, icl_r, Wr1, br1, Wr2, br2, res_o):
    o = out_r[...]
    ne = _dotw(_silu(_dotw(o, Wr1[...]) + br1[...]), Wr2[...]) + br2[...]
    ne = ne * icl_r[...].astype(jnp.float32)
    oh = _onehot_graph(batch_r[...])
    res_o[...] = _dotc0(oh, ne)


def _rb(bs, feat):
    return pl.BlockSpec((bs, feat), lambda i: (i, 0))


def _full(shape):
    nd = len(shape)
    return pl.BlockSpec(shape, lambda i: (0,) * nd)


def _gather_qkv(qT, kT, vT, src, dst):
    mesh = plsc.VectorSubcoreMesh(core_axis_name="c", subcore_axis_name="s")

    @functools.partial(
        pl.kernel,
        out_type=(jax.ShapeDtypeStruct((E_, F_), jnp.float32),) * 3,
        mesh=mesh,
        scratch_types=[
            pltpu.VMEM((_SC_CHUNK,), jnp.int32),
            pltpu.VMEM((_SC_CHUNK,), jnp.int32),
            pltpu.VMEM((_SC_CHUNK, F_), jnp.float32),
            pltpu.VMEM((_SC_CHUNK, F_), jnp.float32),
            pltpu.VMEM((_SC_CHUNK, F_), jnp.float32),
            pltpu.SemaphoreType.DMA,
        ],
    )
    def gk(qT_h, kT_h, vT_h, src_h, dst_h, qg_h, kg_h, vg_h,
           sidx, didx, qb, kb, vb, sem):
        wid = lax.axis_index("s") * 2 + lax.axis_index("c")

        def body(j, carry):
            base = (j * _NW + wid) * _SC_CHUNK
            pltpu.sync_copy(src_h.at[pl.ds(base, _SC_CHUNK)], sidx)
            pltpu.sync_copy(dst_h.at[pl.ds(base, _SC_CHUNK)], didx)
            d1 = pltpu.async_copy(qT_h.at[didx], qb, sem)
            d2 = pltpu.async_copy(kT_h.at[sidx], kb, sem)
            d3 = pltpu.async_copy(vT_h.at[sidx], vb, sem)
            d1.wait()
            d2.wait()
            d3.wait()
            pltpu.sync_copy(qb, qg_h.at[pl.ds(base, _SC_CHUNK)])
            pltpu.sync_copy(kb, kg_h.at[pl.ds(base, _SC_CHUNK)])
            pltpu.sync_copy(vb, vg_h.at[pl.ds(base, _SC_CHUNK)])
            return carry

        lax.fori_loop(0, _SC_ITERS, body, 0)

    return gk(qT, kT, vT, src, dst)


def _scatter_sum(a0, a1, dst, zrows):
    mesh = plsc.VectorSubcoreMesh(core_axis_name="c", subcore_axis_name="s")
    n_iter = E_ // (_SC_CHUNK * 16)

    @functools.partial(
        pl.kernel,
        out_type=jax.ShapeDtypeStruct((2, N_, F_), jnp.float32),
        mesh=mesh,
        scratch_types=[
            pltpu.VMEM((_SC_CHUNK,), jnp.int32),
            pltpu.VMEM((_SC_CHUNK, F_), jnp.float32),
            pltpu.VMEM((_ZROWS, F_), jnp.float32),
            pltpu.VMEM_SHARED((N_, F_), jnp.float32),
        ],
    )
    def sk(a0_h, a1_h, dst_h, zrows_h, hsum_h, didx, mb, rbuf, hs_sh):
        cid = lax.axis_index("c")
        sid = lax.axis_index("s")
        pltpu.sync_copy(zrows_h, rbuf)

        def zbody(j, carry):
            zc = j * 16 + sid

            @pl.when(zc < _NZCH)
            def _():
                r0 = zc * _ZROWS
                pltpu.sync_copy(rbuf, hs_sh.at[pl.ds(r0, _ZROWS)])

            return carry

        lax.fori_loop(0, _ZITER, zbody, 0)
        plsc.subcore_barrier()

        def body(j, carry):
            base = (j * 16 + sid) * _SC_CHUNK
            pltpu.sync_copy(dst_h.at[pl.ds(base, _SC_CHUNK)], didx)

            @pl.when(cid == 0)
            def _():
                pltpu.sync_copy(a0_h.at[pl.ds(base, _SC_CHUNK)], mb)

            @pl.when(cid == 1)
            def _():
                pltpu.sync_copy(a1_h.at[pl.ds(base, _SC_CHUNK)], mb)

            pltpu.sync_copy(mb, hs_sh.at[didx], add=True)
            return carry

        lax.fori_loop(0, n_iter, body, 0)
        plsc.subcore_barrier()

        def fbody(j, carry):
            zc = j * 16 + sid

            @pl.when(zc < _NZCH)
            def _():
                r0 = zc * _ZROWS
                pltpu.sync_copy(hs_sh.at[pl.ds(r0, _ZROWS)], rbuf)
                pltpu.sync_copy(rbuf, hsum_h.at[cid, pl.ds(r0, _ZROWS)])

            return carry

        lax.fori_loop(0, _ZITER, fbody, 0)

    return sk(a0, a1, dst, zrows)


def _edge_precompute(edge_attr, edge_sbf, node_rbf, p):
    (W0, b0), (W1, b1) = p['edgenn']
    convs = p['convs']
    Wek = jnp.stack([c['We_k'] for c in convs])
    bek = jnp.stack([c['be_k'] for c in convs])
    Wev = jnp.stack([c['We_v'] for c in convs])
    bev = jnp.stack([c['be_v'] for c in convs])
    Wg = jnp.stack([c['Wrbf'] for c in convs])
    bg = jnp.stack([c['brbf'] for c in convs])
    Ws = jnp.stack([c['Wsbf'] for c in convs])
    bs = jnp.stack([c['bsbf'] for c in convs])
    grid = E_ // BE
    ef = jax.ShapeDtypeStruct((E_, F_), jnp.float32)
    eh = jax.ShapeDtypeStruct((E_, H_), jnp.float32)
    outs = pl.pallas_call(
        _k1_body,
        grid=(grid,),
        in_specs=[
            _rb(BE, F_), _rb(BE, 112), _rb(BE, 16),
            _full((F_, F_)), _full((1, F_)), _full((F_, F_)), _full((1, F_)),
            _full((3, F_, F_)), _full((3, F_)),
            _full((3, F_, F_)), _full((3, F_)),
            _full((3, 16, F_)), _full((3, F_)),
            _full((3, 112, H_)), _full((3, H_)),
        ],
        out_specs=[_rb(BE, F_)] * 6 + [_rb(BE, F_)] * 3 + [_rb(BE, H_)] * 3,
        out_shape=[ef] * 6 + [ef] * 3 + [eh] * 3,
    )(edge_attr, edge_sbf, node_rbf,
      W0, b0.reshape(1, F_), W1, b1.reshape(1, F_),
      Wek, bek, Wev, bev, Wg, bg, Ws, bs)
    ek = outs[0:3]
    ev = outs[3:6]
    gate = outs[6:9]
    sbfl = outs[9:12]
    return ek, ev, gate, sbfl


def _qkv(h, c):
    nf = jax.ShapeDtypeStruct((N_, F_), jnp.float32)
    return pl.pallas_call(
        _k2_body,
        grid=(N_ // BN,),
        in_specs=[_rb(BN, F_)] + [_full((F_, F_)), _full((1, F_))] * 3,
        out_specs=[_rb(BN, F_)] * 3,
        out_shape=[nf] * 3,
    )(h, c['Wq'], c['bq'].reshape(1, F_), c['Wk'], c['bk'].reshape(1, F_),
      c['Wv'], c['bv'].reshape(1, F_))


def _edge_math(qg, kg, vg, ek, ev, gate, sbfl):
    return pl.pallas_call(
        _k4_body,
        grid=(E_ // BE,),
        in_specs=[_rb(BE, F_)] * 6 + [_rb(BE, H_)],
        out_specs=[_rb(BE, F_), _rb(BE, F_)],
        out_shape=[jax.ShapeDtypeStruct((E_, F_), jnp.float32),
                   jax.ShapeDtypeStruct((E_, F_), jnp.float32)],
    )(qg, kg, vg, ek, ev, gate, sbfl)


def _post(hsum2, res0, batch_col, p, i):
    (Wb1, bb1), (Wb2, bb2) = p['bf_skip'][i]
    Wd, bd = p['dense_bf'][i]
    ((Wa1, ba1), (Wa2, ba2)), ((Wa3, ba3), (Wa4, ba4)) = p['af_skip'][i]
    grid = N_ // BN
    h, S1, S2, cnt = pl.pallas_call(
        _k6a_body,
        grid=(grid,),
        in_specs=[
            pl.BlockSpec((2, BN, F_), lambda i: (0, i, 0)),
            pl.BlockSpec((BN, 1), lambda i: (i, 0)),
        ],
        out_specs=[
            _rb(BN, F_),
            pl.BlockSpec((G_, F_), lambda i: (0, 0)),
            pl.BlockSpec((G_, F_), lambda i: (0, 0)),
            pl.BlockSpec((G_, 1), lambda i: (0, 0)),
        ],
        out_shape=[
            jax.ShapeDtypeStruct((N_, F_), jnp.float32),
            jax.ShapeDtypeStruct((G_, F_), jnp.float32),
            jax.ShapeDtypeStruct((G_, F_), jnp.float32),
            jax.ShapeDtypeStruct((G_, 1), jnp.float32),
        ],
    )(hsum2, batch_col)
    return pl.pallas_call(
        _k6b_body,
        grid=(grid,),
        in_specs=[
            _rb(BN, F_),
            pl.BlockSpec((BN, 1), lambda i: (i, 0)),
            pl.BlockSpec((G_, F_), lambda i: (0, 0)),
            pl.BlockSpec((G_, F_), lambda i: (0, 0)),
            pl.BlockSpec((G_, 1), lambda i: (0, 0)),
            _rb(BN, F_),
        ] + [_full((F_, F_)), _full((1, F_))] * 7,
        out_specs=_rb(BN, F_),
        out_shape=jax.ShapeDtypeStruct((N_, F_), jnp.float32),
    )(h, batch_col, S1, S2, cnt, res0,
      Wb1, bb1.reshape(1, F_), Wb2, bb2.reshape(1, F_),
      Wd, bd.reshape(1, F_),
      Wa1, ba1.reshape(1, F_), Wa2, ba2.reshape(1, F_),
      Wa3, ba3.reshape(1, F_), Wa4, ba4.reshape(1, F_))


def _readout(out, batch_col, icl_col, p):
    (Wr1, br1), (Wr2, br2) = p['readout']
    return pl.pallas_call(
        _k7_body,
        out_shape=jax.ShapeDtypeStruct((G_, 1), jnp.float32),
    )(out, batch_col, icl_col, Wr1, br1.reshape(1, F_),
      Wr2, br2.reshape(1, 1))


def kernel(x, edge_index, edge_attr, edge_sbf, node_rbf, batch, is_cleave,
           num_graphs, params):
    src = edge_index[0]
    dst = edge_index[1]
    batch_col = batch.reshape(N_, 1)
    icl_col = is_cleave.reshape(N_, 1)
    zrows = jnp.zeros((_ZROWS, F_), jnp.float32)

    ek, ev, gate, sbfl = _edge_precompute(edge_attr, edge_sbf, node_rbf, params)

    out = x
    for i in range(3):
        c = params['convs'][i]
        q, kT, vT = _qkv(out, c)
        qg, kg, vg = _gather_qkv(q, kT, vT, src, dst)
        a0, a1 = _edge_math(qg, kg, vg, ek[i], ev[i], gate[i], sbfl[i])
        hsum2 = _scatter_sum(a0, a1, dst, zrows)
        out = _post(hsum2, out, batch_col, params, i)

    res = _readout(out, batch_col, icl_col, params)
    return res.reshape(-1)


# speedup vs baseline: 26.4712x; 1.5291x over previous
import functools

import jax
import jax.numpy as jnp
from jax import lax
from jax.experimental import pallas as pl
from jax.experimental.pallas import tpu as pltpu
from jax.experimental.pallas import tpu_sc as plsc

N_ = 10000
E_ = 160000
F_ = 128
H_ = 8
CH_ = 16
G_ = 128
EPS = 1e-8

BE = 2000
BN = 2000

_NW = 32
_SC_CHUNK = 40
_SC_ITERS = E_ // (_SC_CHUNK * _NW)
_ZROWS = 80
_NZCH = N_ // _ZROWS
_ZITER = -(-_NZCH // 16)


def _silu(t):
    return t * jax.nn.sigmoid(t)


def _dot(a, b):
    return jnp.dot(a, b, precision=lax.Precision.HIGHEST)


def _dotw(a, b):
    return jnp.dot(a, b, precision=lax.Precision.DEFAULT)


def _dotc0(a, b):
    return lax.dot_general(a, b, (((0,), (0,)), ((), ())),
                           precision=lax.Precision.HIGHEST)


def _head_mats():
    hsel = (lax.broadcasted_iota(jnp.int32, (F_, H_), 0) // CH_
            == lax.broadcasted_iota(jnp.int32, (F_, H_), 1)).astype(jnp.float32)
    hexp = (lax.broadcasted_iota(jnp.int32, (H_, F_), 1) // CH_
            == lax.broadcasted_iota(jnp.int32, (H_, F_), 0)).astype(jnp.float32)
    return hsel, hexp


def _onehot_graph(batch_col):
    return (batch_col == lax.broadcasted_iota(jnp.int32, (N_, G_), 1)
            ).astype(jnp.float32)


def _k1_body(ea_r, sbf_r, rbf_r, W0r, b0r, W1r, b1r,
             Wekr, bekr, Wevr, bevr, Wgr, bgr, Wsr, bsr, *outs):
    a = _dotw(_silu(_dotw(ea_r[...], W0r[...]) + b0r[...]), W1r[...]) + b1r[...]
    sbf = sbf_r[...]
    rbf = rbf_r[...]
    Wek, bek = Wekr[...], bekr[...]
    Wev, bev = Wevr[...], bevr[...]
    Wg, bg = Wgr[...], bgr[...]
    Ws, bs = Wsr[...], bsr[...]
    for i in range(3):
        outs[i][...] = _dotw(a, Wek[i]) + bek[i][None, :]
        outs[3 + i][...] = _dotw(a, Wev[i]) + bev[i][None, :]
        outs[6 + i][...] = _dotw(rbf, Wg[i]) + bg[i][None, :]
        outs[9 + i][...] = _dotw(sbf, Ws[i]) + bs[i][None, :]


def _k2_body(x_r, Wq, bq, Wk, bk, Wv, bv, q_o, k_o, v_o):
    xv = x_r[...]
    q_o[...] = _dotw(xv, Wq[...]) + bq[...]
    k_o[...] = _dotw(xv, Wk[...]) + bk[...]
    v_o[...] = _dotw(xv, Wv[...]) + bv[...]


def _k4_body(qg_r, kg_r, vg_r, ek_r, ev_r, g_r, s_r, a0_o, a1_o):
    hsel, hexp = _head_mats()
    qg = qg_r[...]
    kk = kg_r[...] + ek_r[...]
    logits = _dot(qg * kk, hsel) * 0.25 + s_r[...]
    ehf = _dot(jnp.exp(logits), hexp)
    m = ehf * (vg_r[...] + ev_r[...]) * g_r[...]
    a0_o[...] = jnp.concatenate([m[:, :64], ehf[:, :64]], axis=1)
    a1_o[...] = jnp.concatenate([m[:, 64:], ehf[:, 64:]], axis=1)


def _onehot_block(batch_col):
    return (batch_col == lax.broadcasted_iota(jnp.int32, (BN, G_), 1)
            ).astype(jnp.float32)


def _k6a_body(hs_r, batch_r, h_o, S1_o, S2_o, cnt_o):
    i = pl.program_id(0)
    hs = hs_r[...]
    h = jnp.concatenate(
        [hs[0, :, :64] / (hs[0, :, 64:] + 1e-16),
         hs[1, :, :64] / (hs[1, :, 64:] + 1e-16)], axis=1)
    h_o[...] = h
    oh = _onehot_block(batch_r[...])
    ones = jnp.full((BN, 1), 1.0, jnp.float32)

    @pl.when(i == 0)
    def _():
        S1_o[...] = jnp.zeros_like(S1_o)
        S2_o[...] = jnp.zeros_like(S2_o)
        cnt_o[...] = jnp.zeros_like(cnt_o)

    S1_o[...] += _dotc0(oh, h)
    S2_o[...] += _dotc0(oh, h * h)
    cnt_o[...] += _dotc0(oh, ones)


def _k6b_body(h_r, batch_r, S1_r, S2_r, cnt_r, res0_r,
              Wb1, bb1, Wb2, bb2, Wd, bd,
              Wa1, ba1, Wa2, ba2, Wa3, ba3, Wa4, ba4, out_o):
    n = cnt_r[...] * jnp.float32(F_) + 1e-12
    mean = jnp.sum(S1_r[...], axis=1, keepdims=True) / n
    var = jnp.sum(S2_r[...], axis=1, keepdims=True) / n - mean * mean
    sd = jnp.sqrt(var + EPS) + EPS
    oh = _onehot_block(batch_r[...])
    h = (h_r[...] - _dot(oh, mean)) / _dot(oh, sd)
    h = h + _silu(_dotw(_silu(_dotw(h, Wb1[...]) + bb1[...]), Wb2[...]) + bb2[...])
    h = _silu(_dotw(h, Wd[...]) + bd[...])
    h = h + res0_r[...]
    h = h + _silu(_dotw(_silu(_dotw(h, Wa1[...]) + ba1[...]), Wa2[...]) + ba2[...])
    h = h + _silu(_dotw(_silu(_dotw(h, Wa3[...]) + ba3[...]), Wa4[...]) + ba4[...])
    out_o[...] = h


def _k7_body(out_r, batch_r, icl_r, Wr1, br1, Wr2, br2, res_o):
    o = out_r[...]
    ne = _dotw(_silu(_dotw(o, Wr1[...]) + br1[...]), Wr2[...]) + br2[...]
    ne = ne * icl_r[...].astype(jnp.float32)
    oh = _onehot_graph(batch_r[...])
    res_o[...] = _dotc0(oh, ne)


def _rb(bs, feat):
    return pl.BlockSpec((bs, feat), lambda i: (i, 0))


def _full(shape):
    nd = len(shape)
    return pl.BlockSpec(shape, lambda i: (0,) * nd)


_NB = 5


def _gather_qkv(qT, kT, vT, src, dst):
    mesh = plsc.VectorSubcoreMesh(core_axis_name="c", subcore_axis_name="s")
    C = _SC_CHUNK
    nch = _SC_ITERS

    scratch = (
        [pltpu.VMEM((C,), jnp.int32)] * _NB
        + [pltpu.VMEM((C,), jnp.int32)] * _NB
        + [pltpu.VMEM((C, F_), jnp.float32)] * _NB
        + [pltpu.VMEM((C, F_), jnp.float32)] * _NB
        + [pltpu.VMEM((C, F_), jnp.float32)] * _NB
        + [pltpu.SemaphoreType.DMA] * (2 * _NB)
    )

    @functools.partial(
        pl.kernel,
        out_type=(jax.ShapeDtypeStruct((E_, F_), jnp.float32),) * 3,
        mesh=mesh,
        scratch_types=scratch,
    )
    def gk(qT_h, kT_h, vT_h, src_h, dst_h, qg_h, kg_h, vg_h, *scr):
        sidx = scr[0:_NB]
        didx = scr[_NB:2 * _NB]
        qb = scr[2 * _NB:3 * _NB]
        kb = scr[3 * _NB:4 * _NB]
        vb = scr[4 * _NB:5 * _NB]
        gsem = scr[5 * _NB:6 * _NB]
        wsem = scr[6 * _NB:7 * _NB]
        wid = lax.axis_index("s") * 2 + lax.axis_index("c")

        def base(t):
            return (t * _NW + wid) * C

        def fire(t, b):
            pltpu.sync_copy(src_h.at[pl.ds(base(t), C)], sidx[b])
            pltpu.sync_copy(dst_h.at[pl.ds(base(t), C)], didx[b])
            pltpu.async_copy(qT_h.at[didx[b]], qb[b], gsem[b])
            pltpu.async_copy(kT_h.at[sidx[b]], kb[b], gsem[b])
            pltpu.async_copy(vT_h.at[sidx[b]], vb[b], gsem[b])

        def drain_g(b):
            pltpu.make_async_copy(qT_h.at[didx[b]], qb[b], gsem[b]).wait()
            pltpu.make_async_copy(kT_h.at[sidx[b]], kb[b], gsem[b]).wait()
            pltpu.make_async_copy(vT_h.at[sidx[b]], vb[b], gsem[b]).wait()

        def fire_wb(t, b):
            pltpu.async_copy(qb[b], qg_h.at[pl.ds(base(t), C)], wsem[b])
            pltpu.async_copy(kb[b], kg_h.at[pl.ds(base(t), C)], wsem[b])
            pltpu.async_copy(vb[b], vg_h.at[pl.ds(base(t), C)], wsem[b])

        def drain_wb(t, b):
            pltpu.make_async_copy(qb[b], qg_h.at[pl.ds(base(t), C)], wsem[b]).wait()
            pltpu.make_async_copy(kb[b], kg_h.at[pl.ds(base(t), C)], wsem[b]).wait()
            pltpu.make_async_copy(vb[b], vg_h.at[pl.ds(base(t), C)], wsem[b]).wait()

        fire(0, 0)
        fire(1, 1)

        def step(s, carry):
            for b in range(_NB):
                t = s * _NB + b
                tb = (b + 2) % _NB

                @pl.when(t >= 3)
                def _():
                    drain_wb(t - 3, tb)

                @pl.when(t + 2 < nch)
                def _():
                    fire(t + 2, tb)

                drain_g(b)
                fire_wb(t, b)
            return carry

        lax.fori_loop(0, nch // _NB, step, 0)
        drain_wb(nch - 3, (nch - 3) % _NB)
        drain_wb(nch - 2, (nch - 2) % _NB)
        drain_wb(nch - 1, (nch - 1) % _NB)

    return gk(qT, kT, vT, src, dst)


def _scatter_sum(a0, a1, dst, zrows):
    mesh = plsc.VectorSubcoreMesh(core_axis_name="c", subcore_axis_name="s")
    n_iter = E_ // (_SC_CHUNK * 16)

    C = _SC_CHUNK
    scratch = (
        [pltpu.VMEM((C,), jnp.int32)] * _NB
        + [pltpu.VMEM((C, F_), jnp.float32)] * _NB
        + [pltpu.SemaphoreType.DMA] * (2 * _NB)
        + [
            pltpu.VMEM((_ZROWS, F_), jnp.float32),
            pltpu.VMEM_SHARED((N_, F_), jnp.float32),
        ]
    )

    @functools.partial(
        pl.kernel,
        out_type=jax.ShapeDtypeStruct((2, N_, F_), jnp.float32),
        mesh=mesh,
        scratch_types=scratch,
    )
    def sk(a0_h, a1_h, dst_h, zrows_h, hsum_h, *scr):
        didx = scr[0:_NB]
        mb = scr[_NB:2 * _NB]
        rsem = scr[2 * _NB:3 * _NB]
        ssem = scr[3 * _NB:4 * _NB]
        rbuf, hs_sh = scr[4 * _NB], scr[4 * _NB + 1]
        cid = lax.axis_index("c")
        sid = lax.axis_index("s")
        pltpu.sync_copy(zrows_h, rbuf)

        def zbody(j, carry):
            zc = j * 16 + sid

            @pl.when(zc < _NZCH)
            def _():
                r0 = zc * _ZROWS
                pltpu.sync_copy(rbuf, hs_sh.at[pl.ds(r0, _ZROWS)])

            return carry

        lax.fori_loop(0, _ZITER, zbody, 0)
        plsc.subcore_barrier()

        def base(t):
            return (t * 16 + sid) * C

        def fire_r(t, b):
            pltpu.async_copy(dst_h.at[pl.ds(base(t), C)], didx[b], rsem[b])

            @pl.when(cid == 0)
            def _():
                pltpu.async_copy(a0_h.at[pl.ds(base(t), C)], mb[b], rsem[b])

            @pl.when(cid == 1)
            def _():
                pltpu.async_copy(a1_h.at[pl.ds(base(t), C)], mb[b], rsem[b])

        def drain_r(t, b):
            pltpu.make_async_copy(dst_h.at[pl.ds(base(t), C)], didx[b],
                                  rsem[b]).wait()
            pltpu.make_async_copy(a0_h.at[pl.ds(base(t), C)], mb[b],
                                  rsem[b]).wait()

        def fire_s(b):
            pltpu.async_copy(mb[b], hs_sh.at[didx[b]], ssem[b], add=True)

        def drain_s(b):
            pltpu.make_async_copy(mb[b], hs_sh.at[didx[b]], ssem[b]).wait()

        fire_r(0, 0)
        fire_r(1, 1)

        def body(s, carry):
            for b in range(_NB):
                t = s * _NB + b
                tb = (b + 2) % _NB

                @pl.when(t >= 3)
                def _():
                    drain_s(tb)

                @pl.when(t + 2 < n_iter)
                def _():
                    fire_r(t + 2, tb)

                drain_r(t, b)
                fire_s(b)
            return carry

        lax.fori_loop(0, n_iter // _NB, body, 0)
        drain_s((n_iter - 3) % _NB)
        drain_s((n_iter - 2) % _NB)
        drain_s((n_iter - 1) % _NB)
        plsc.subcore_barrier()

        def fbody(j, carry):
            zc = j * 16 + sid

            @pl.when(zc < _NZCH)
            def _():
                r0 = zc * _ZROWS
                pltpu.sync_copy(hs_sh.at[pl.ds(r0, _ZROWS)], rbuf)
                pltpu.sync_copy(rbuf, hsum_h.at[cid, pl.ds(r0, _ZROWS)])

            return carry

        lax.fori_loop(0, _ZITER, fbody, 0)

    return sk(a0, a1, dst, zrows)


def _edge_precompute(edge_attr, edge_sbf, node_rbf, p):
    (W0, b0), (W1, b1) = p['edgenn']
    convs = p['convs']
    Wek = jnp.stack([c['We_k'] for c in convs])
    bek = jnp.stack([c['be_k'] for c in convs])
    Wev = jnp.stack([c['We_v'] for c in convs])
    bev = jnp.stack([c['be_v'] for c in convs])
    Wg = jnp.stack([c['Wrbf'] for c in convs])
    bg = jnp.stack([c['brbf'] for c in convs])
    Ws = jnp.stack([c['Wsbf'] for c in convs])
    bs = jnp.stack([c['bsbf'] for c in convs])
    grid = E_ // BE
    ef = jax.ShapeDtypeStruct((E_, F_), jnp.float32)
    eh = jax.ShapeDtypeStruct((E_, H_), jnp.float32)
    outs = pl.pallas_call(
        _k1_body,
        grid=(grid,),
        in_specs=[
            _rb(BE, F_), _rb(BE, 112), _rb(BE, 16),
            _full((F_, F_)), _full((1, F_)), _full((F_, F_)), _full((1, F_)),
            _full((3, F_, F_)), _full((3, F_)),
            _full((3, F_, F_)), _full((3, F_)),
            _full((3, 16, F_)), _full((3, F_)),
            _full((3, 112, H_)), _full((3, H_)),
        ],
        out_specs=[_rb(BE, F_)] * 6 + [_rb(BE, F_)] * 3 + [_rb(BE, H_)] * 3,
        out_shape=[ef] * 6 + [ef] * 3 + [eh] * 3,
    )(edge_attr, edge_sbf, node_rbf,
      W0, b0.reshape(1, F_), W1, b1.reshape(1, F_),
      Wek, bek, Wev, bev, Wg, bg, Ws, bs)
    ek = outs[0:3]
    ev = outs[3:6]
    gate = outs[6:9]
    sbfl = outs[9:12]
    return ek, ev, gate, sbfl


def _qkv(h, c):
    nf = jax.ShapeDtypeStruct((N_, F_), jnp.float32)
    return pl.pallas_call(
        _k2_body,
        grid=(N_ // BN,),
        in_specs=[_rb(BN, F_)] + [_full((F_, F_)), _full((1, F_))] * 3,
        out_specs=[_rb(BN, F_)] * 3,
        out_shape=[nf] * 3,
    )(h, c['Wq'], c['bq'].reshape(1, F_), c['Wk'], c['bk'].reshape(1, F_),
      c['Wv'], c['bv'].reshape(1, F_))


def _edge_math(qg, kg, vg, ek, ev, gate, sbfl):
    return pl.pallas_call(
        _k4_body,
        grid=(E_ // BE,),
        in_specs=[_rb(BE, F_)] * 6 + [_rb(BE, H_)],
        out_specs=[_rb(BE, F_), _rb(BE, F_)],
        out_shape=[jax.ShapeDtypeStruct((E_, F_), jnp.float32),
                   jax.ShapeDtypeStruct((E_, F_), jnp.float32)],
    )(qg, kg, vg, ek, ev, gate, sbfl)


def _post(hsum2, res0, batch_col, p, i):
    (Wb1, bb1), (Wb2, bb2) = p['bf_skip'][i]
    Wd, bd = p['dense_bf'][i]
    ((Wa1, ba1), (Wa2, ba2)), ((Wa3, ba3), (Wa4, ba4)) = p['af_skip'][i]
    grid = N_ // BN
    h, S1, S2, cnt = pl.pallas_call(
        _k6a_body,
        grid=(grid,),
        in_specs=[
            pl.BlockSpec((2, BN, F_), lambda i: (0, i, 0)),
            pl.BlockSpec((BN, 1), lambda i: (i, 0)),
        ],
        out_specs=[
            _rb(BN, F_),
            pl.BlockSpec((G_, F_), lambda i: (0, 0)),
            pl.BlockSpec((G_, F_), lambda i: (0, 0)),
            pl.BlockSpec((G_, 1), lambda i: (0, 0)),
        ],
        out_shape=[
            jax.ShapeDtypeStruct((N_, F_), jnp.float32),
            jax.ShapeDtypeStruct((G_, F_), jnp.float32),
            jax.ShapeDtypeStruct((G_, F_), jnp.float32),
            jax.ShapeDtypeStruct((G_, 1), jnp.float32),
        ],
    )(hsum2, batch_col)
    return pl.pallas_call(
        _k6b_body,
        grid=(grid,),
        in_specs=[
            _rb(BN, F_),
            pl.BlockSpec((BN, 1), lambda i: (i, 0)),
            pl.BlockSpec((G_, F_), lambda i: (0, 0)),
            pl.BlockSpec((G_, F_), lambda i: (0, 0)),
            pl.BlockSpec((G_, 1), lambda i: (0, 0)),
            _rb(BN, F_),
        ] + [_full((F_, F_)), _full((1, F_))] * 7,
        out_specs=_rb(BN, F_),
        out_shape=jax.ShapeDtypeStruct((N_, F_), jnp.float32),
    )(h, batch_col, S1, S2, cnt, res0,
      Wb1, bb1.reshape(1, F_), Wb2, bb2.reshape(1, F_),
      Wd, bd.reshape(1, F_),
      Wa1, ba1.reshape(1, F_), Wa2, ba2.reshape(1, F_),
      Wa3, ba3.reshape(1, F_), Wa4, ba4.reshape(1, F_))


def _readout(out, batch_col, icl_col, p):
    (Wr1, br1), (Wr2, br2) = p['readout']
    return pl.pallas_call(
        _k7_body,
        out_shape=jax.ShapeDtypeStruct((G_, 1), jnp.float32),
    )(out, batch_col, icl_col, Wr1, br1.reshape(1, F_),
      Wr2, br2.reshape(1, 1))


def kernel(x, edge_index, edge_attr, edge_sbf, node_rbf, batch, is_cleave,
           num_graphs, params):
    src = edge_index[0]
    dst = edge_index[1]
    batch_col = batch.reshape(N_, 1)
    icl_col = is_cleave.reshape(N_, 1)
    zrows = jnp.zeros((_ZROWS, F_), jnp.float32)

    ek, ev, gate, sbfl = _edge_precompute(edge_attr, edge_sbf, node_rbf, params)

    out = x
    for i in range(3):
        c = params['convs'][i]
        q, kT, vT = _qkv(out, c)
        qg, kg, vg = _gather_qkv(q, kT, vT, src, dst)
        a0, a1 = _edge_math(qg, kg, vg, ek[i], ev[i], gate[i], sbfl[i])
        hsum2 = _scatter_sum(a0, a1, dst, zrows)
        out = _post(hsum2, out, batch_col, params, i)

    res = _readout(out, batch_col, icl_col, params)
    return res.reshape(-1)


# speedup vs baseline: 26.4904x; 1.0007x over previous
import functools

import jax
import jax.numpy as jnp
from jax import lax
from jax.experimental import pallas as pl
from jax.experimental.pallas import tpu as pltpu
from jax.experimental.pallas import tpu_sc as plsc

N_ = 10000
E_ = 160000
F_ = 128
H_ = 8
CH_ = 16
G_ = 128
EPS = 1e-8

BE = 2000
BN = 2000

_NW = 32
_SC_CHUNK = 40
_SC_ITERS = E_ // (_SC_CHUNK * _NW)
_ZROWS = 80
_NZCH = N_ // _ZROWS
_ZITER = -(-_NZCH // 16)


def _silu(t):
    return t * jax.nn.sigmoid(t)


def _dot(a, b):
    return jnp.dot(a, b, precision=lax.Precision.HIGHEST)


def _dotw(a, b):
    return jnp.dot(a, b, precision=lax.Precision.DEFAULT)


def _dotc0(a, b):
    return lax.dot_general(a, b, (((0,), (0,)), ((), ())),
                           precision=lax.Precision.HIGHEST)


def _head_mats():
    hsel = (lax.broadcasted_iota(jnp.int32, (F_, H_), 0) // CH_
            == lax.broadcasted_iota(jnp.int32, (F_, H_), 1)).astype(jnp.float32)
    hexp = (lax.broadcasted_iota(jnp.int32, (H_, F_), 1) // CH_
            == lax.broadcasted_iota(jnp.int32, (H_, F_), 0)).astype(jnp.float32)
    return hsel, hexp


def _onehot_graph(batch_col):
    return (batch_col == lax.broadcasted_iota(jnp.int32, (N_, G_), 1)
            ).astype(jnp.float32)


def _k1_body(ea_r, sbf_r, rbf_r, W0r, b0r, W1r, b1r,
             Wekr, bekr, Wevr, bevr, Wgr, bgr, Wsr, bsr, *outs):
    a = _dotw(_silu(_dotw(ea_r[...], W0r[...]) + b0r[...]), W1r[...]) + b1r[...]
    sbf = sbf_r[...]
    rbf = rbf_r[...]
    Wek, bek = Wekr[...], bekr[...]
    Wev, bev = Wevr[...], bevr[...]
    Wg, bg = Wgr[...], bgr[...]
    Ws, bs = Wsr[...], bsr[...]
    for i in range(3):
        outs[i][...] = _dotw(a, Wek[i]) + bek[i][None, :]
        outs[3 + i][...] = _dotw(a, Wev[i]) + bev[i][None, :]
        outs[6 + i][...] = _dotw(rbf, Wg[i]) + bg[i][None, :]
        outs[9 + i][...] = _dotw(sbf, Ws[i]) + bs[i][None, :]


def _k2_body(x_r, Wq, bq, Wk, bk, Wv, bv, q_o, kv_o):
    xv = x_r[...]
    q_o[...] = _dotw(xv, Wq[...]) + bq[...]
    kv_o[...] = jnp.concatenate(
        [_dotw(xv, Wk[...]) + bk[...], _dotw(xv, Wv[...]) + bv[...]], axis=1)


def _k4_body(qg_r, kvg_r, ek_r, ev_r, g_r, s_r, a0_o, a1_o):
    hsel, hexp = _head_mats()
    qg = qg_r[...]
    kvg = kvg_r[...]
    kk = kvg[:, :F_] + ek_r[...]
    logits = _dot(qg * kk, hsel) * 0.25 + s_r[...]
    ehf = _dot(jnp.exp(logits), hexp)
    m = ehf * (kvg[:, F_:] + ev_r[...]) * g_r[...]
    a0_o[...] = jnp.concatenate([m[:, :64], ehf[:, :64]], axis=1)
    a1_o[...] = jnp.concatenate([m[:, 64:], ehf[:, 64:]], axis=1)


def _onehot_block(batch_col):
    return (batch_col == lax.broadcasted_iota(jnp.int32, (BN, G_), 1)
            ).astype(jnp.float32)


def _k6a_body(hs_r, batch_r, h_o, S1_o, S2_o, cnt_o):
    i = pl.program_id(0)
    hs = hs_r[...]
    h = jnp.concatenate(
        [hs[0, :, :64] / (hs[0, :, 64:] + 1e-16),
         hs[1, :, :64] / (hs[1, :, 64:] + 1e-16)], axis=1)
    h_o[...] = h
    oh = _onehot_block(batch_r[...])
    ones = jnp.full((BN, 1), 1.0, jnp.float32)

    @pl.when(i == 0)
    def _():
        S1_o[...] = jnp.zeros_like(S1_o)
        S2_o[...] = jnp.zeros_like(S2_o)
        cnt_o[...] = jnp.zeros_like(cnt_o)

    S1_o[...] += _dotc0(oh, h)
    S2_o[...] += _dotc0(oh, h * h)
    cnt_o[...] += _dotc0(oh, ones)


def _k6b_body(h_r, batch_r, S1_r, S2_r, cnt_r, res0_r,
              Wb1, bb1, Wb2, bb2, Wd, bd,
              Wa1, ba1, Wa2, ba2, Wa3, ba3, Wa4, ba4, out_o):
    n = cnt_r[...] * jnp.float32(F_) + 1e-12
    mean = jnp.sum(S1_r[...], axis=1, keepdims=True) / n
    var = jnp.sum(S2_r[...], axis=1, keepdims=True) / n - mean * mean
    sd = jnp.sqrt(var + EPS) + EPS
    oh = _onehot_block(batch_r[...])
    h = (h_r[...] - _dot(oh, mean)) / _dot(oh, sd)
    h = h + _silu(_dotw(_silu(_dotw(h, Wb1[...]) + bb1[...]), Wb2[...]) + bb2[...])
    h = _silu(_dotw(h, Wd[...]) + bd[...])
    h = h + res0_r[...]
    h = h + _silu(_dotw(_silu(_dotw(h, Wa1[...]) + ba1[...]), Wa2[...]) + ba2[...])
    h = h + _silu(_dotw(_silu(_dotw(h, Wa3[...]) + ba3[...]), Wa4[...]) + ba4[...])
    out_o[...] = h


def _k7_body(out_r, batch_r, icl_r, Wr1, br1, Wr2, br2, res_o):
    o = out_r[...]
    ne = _dotw(_silu(_dotw(o, Wr1[...]) + br1[...]), Wr2[...]) + br2[...]
    ne = ne * icl_r[...].astype(jnp.float32)
    oh = _onehot_graph(batch_r[...])
    res_o[...] = _dotc0(oh, ne)


def _rb(bs, feat):
    return pl.BlockSpec((bs, feat), lambda i: (i, 0))


def _full(shape):
    nd = len(shape)
    return pl.BlockSpec(shape, lambda i: (0,) * nd)


_NB = 5


def _gather_qkv(qT, kvT, src, dst):
    mesh = plsc.VectorSubcoreMesh(core_axis_name="c", subcore_axis_name="s")
    C = _SC_CHUNK
    nch = _SC_ITERS

    scratch = (
        [pltpu.VMEM((C,), jnp.int32)] * _NB
        + [pltpu.VMEM((C,), jnp.int32)] * _NB
        + [pltpu.VMEM((C, F_), jnp.float32)] * _NB
        + [pltpu.VMEM((C, 2 * F_), jnp.float32)] * _NB
        + [pltpu.SemaphoreType.DMA] * (2 * _NB)
    )

    @functools.partial(
        pl.kernel,
        out_type=(jax.ShapeDtypeStruct((E_, F_), jnp.float32),
                  jax.ShapeDtypeStruct((E_, 2 * F_), jnp.float32)),
        mesh=mesh,
        scratch_types=scratch,
    )
    def gk(qT_h, kvT_h, src_h, dst_h, qg_h, kvg_h, *scr):
        sidx = scr[0:_NB]
        didx = scr[_NB:2 * _NB]
        qb = scr[2 * _NB:3 * _NB]
        kvb = scr[3 * _NB:4 * _NB]
        gsem = scr[4 * _NB:5 * _NB]
        wsem = scr[5 * _NB:6 * _NB]
        wid = lax.axis_index("s") * 2 + lax.axis_index("c")

        def base(t):
            return (t * _NW + wid) * C

        def fire(t, b):
            pltpu.sync_copy(src_h.at[pl.ds(base(t), C)], sidx[b])
            pltpu.sync_copy(dst_h.at[pl.ds(base(t), C)], didx[b])
            pltpu.async_copy(qT_h.at[didx[b]], qb[b], gsem[b])
            pltpu.async_copy(kvT_h.at[sidx[b]], kvb[b], gsem[b])

        def drain_g(b):
            pltpu.make_async_copy(qT_h.at[didx[b]], qb[b], gsem[b]).wait()
            pltpu.make_async_copy(kvT_h.at[sidx[b]], kvb[b], gsem[b]).wait()

        def fire_wb(t, b):
            pltpu.async_copy(qb[b], qg_h.at[pl.ds(base(t), C)], wsem[b])
            pltpu.async_copy(kvb[b], kvg_h.at[pl.ds(base(t), C)], wsem[b])

        def drain_wb(t, b):
            pltpu.make_async_copy(qb[b], qg_h.at[pl.ds(base(t), C)], wsem[b]).wait()
            pltpu.make_async_copy(kvb[b], kvg_h.at[pl.ds(base(t), C)], wsem[b]).wait()

        fire(0, 0)
        fire(1, 1)

        def step(s, carry):
            for b in range(_NB):
                t = s * _NB + b
                tb = (b + 2) % _NB

                @pl.when(t >= 3)
                def _():
                    drain_wb(t - 3, tb)

                @pl.when(t + 2 < nch)
                def _():
                    fire(t + 2, tb)

                drain_g(b)
                fire_wb(t, b)
            return carry

        lax.fori_loop(0, nch // _NB, step, 0)
        drain_wb(nch - 3, (nch - 3) % _NB)
        drain_wb(nch - 2, (nch - 2) % _NB)
        drain_wb(nch - 1, (nch - 1) % _NB)

    return gk(qT, kvT, src, dst)


def _scatter_sum(a0, a1, dst, zrows):
    mesh = plsc.VectorSubcoreMesh(core_axis_name="c", subcore_axis_name="s")
    n_iter = E_ // (_SC_CHUNK * 16)

    C = _SC_CHUNK
    scratch = (
        [pltpu.VMEM((C,), jnp.int32)] * _NB
        + [pltpu.VMEM((C, F_), jnp.float32)] * _NB
        + [pltpu.SemaphoreType.DMA] * (2 * _NB)
        + [
            pltpu.VMEM((_ZROWS, F_), jnp.float32),
            pltpu.VMEM_SHARED((N_, F_), jnp.float32),
        ]
    )

    @functools.partial(
        pl.kernel,
        out_type=jax.ShapeDtypeStruct((2, N_, F_), jnp.float32),
        mesh=mesh,
        scratch_types=scratch,
    )
    def sk(a0_h, a1_h, dst_h, zrows_h, hsum_h, *scr):
        didx = scr[0:_NB]
        mb = scr[_NB:2 * _NB]
        rsem = scr[2 * _NB:3 * _NB]
        ssem = scr[3 * _NB:4 * _NB]
        rbuf, hs_sh = scr[4 * _NB], scr[4 * _NB + 1]
        cid = lax.axis_index("c")
        sid = lax.axis_index("s")
        pltpu.sync_copy(zrows_h, rbuf)

        def zbody(j, carry):
            zc = j * 16 + sid

            @pl.when(zc < _NZCH)
            def _():
                r0 = zc * _ZROWS
                pltpu.sync_copy(rbuf, hs_sh.at[pl.ds(r0, _ZROWS)])

            return carry

        lax.fori_loop(0, _ZITER, zbody, 0)
        plsc.subcore_barrier()

        def base(t):
            return (t * 16 + sid) * C

        def fire_r(t, b):
            pltpu.async_copy(dst_h.at[pl.ds(base(t), C)], didx[b], rsem[b])

            @pl.when(cid == 0)
            def _():
                pltpu.async_copy(a0_h.at[pl.ds(base(t), C)], mb[b], rsem[b])

            @pl.when(cid == 1)
            def _():
                pltpu.async_copy(a1_h.at[pl.ds(base(t), C)], mb[b], rsem[b])

        def drain_r(t, b):
            pltpu.make_async_copy(dst_h.at[pl.ds(base(t), C)], didx[b],
                                  rsem[b]).wait()
            pltpu.make_async_copy(a0_h.at[pl.ds(base(t), C)], mb[b],
                                  rsem[b]).wait()

        def fire_s(b):
            pltpu.async_copy(mb[b], hs_sh.at[didx[b]], ssem[b], add=True)

        def drain_s(b):
            pltpu.make_async_copy(mb[b], hs_sh.at[didx[b]], ssem[b]).wait()

        fire_r(0, 0)
        fire_r(1, 1)

        def body(s, carry):
            for b in range(_NB):
                t = s * _NB + b
                tb = (b + 2) % _NB

                @pl.when(t >= 3)
                def _():
                    drain_s(tb)

                @pl.when(t + 2 < n_iter)
                def _():
                    fire_r(t + 2, tb)

                drain_r(t, b)
                fire_s(b)
            return carry

        lax.fori_loop(0, n_iter // _NB, body, 0)
        drain_s((n_iter - 3) % _NB)
        drain_s((n_iter - 2) % _NB)
        drain_s((n_iter - 1) % _NB)
        plsc.subcore_barrier()

        def fbody(j, carry):
            zc = j * 16 + sid

            @pl.when(zc < _NZCH)
            def _():
                r0 = zc * _ZROWS
                pltpu.sync_copy(hs_sh.at[pl.ds(r0, _ZROWS)], rbuf)
                pltpu.sync_copy(rbuf, hsum_h.at[cid, pl.ds(r0, _ZROWS)])

            return carry

        lax.fori_loop(0, _ZITER, fbody, 0)

    return sk(a0, a1, dst, zrows)


def _edge_precompute(edge_attr, edge_sbf, node_rbf, p):
    (W0, b0), (W1, b1) = p['edgenn']
    convs = p['convs']
    Wek = jnp.stack([c['We_k'] for c in convs])
    bek = jnp.stack([c['be_k'] for c in convs])
    Wev = jnp.stack([c['We_v'] for c in convs])
    bev = jnp.stack([c['be_v'] for c in convs])
    Wg = jnp.stack([c['Wrbf'] for c in convs])
    bg = jnp.stack([c['brbf'] for c in convs])
    Ws = jnp.stack([c['Wsbf'] for c in convs])
    bs = jnp.stack([c['bsbf'] for c in convs])
    grid = E_ // BE
    ef = jax.ShapeDtypeStruct((E_, F_), jnp.float32)
    eh = jax.ShapeDtypeStruct((E_, H_), jnp.float32)
    outs = pl.pallas_call(
        _k1_body,
        grid=(grid,),
        in_specs=[
            _rb(BE, F_), _rb(BE, 112), _rb(BE, 16),
            _full((F_, F_)), _full((1, F_)), _full((F_, F_)), _full((1, F_)),
            _full((3, F_, F_)), _full((3, F_)),
            _full((3, F_, F_)), _full((3, F_)),
            _full((3, 16, F_)), _full((3, F_)),
            _full((3, 112, H_)), _full((3, H_)),
        ],
        out_specs=[_rb(BE, F_)] * 6 + [_rb(BE, F_)] * 3 + [_rb(BE, H_)] * 3,
        out_shape=[ef] * 6 + [ef] * 3 + [eh] * 3,
    )(edge_attr, edge_sbf, node_rbf,
      W0, b0.reshape(1, F_), W1, b1.reshape(1, F_),
      Wek, bek, Wev, bev, Wg, bg, Ws, bs)
    ek = outs[0:3]
    ev = outs[3:6]
    gate = outs[6:9]
    sbfl = outs[9:12]
    return ek, ev, gate, sbfl


def _qkv(h, c):
    return pl.pallas_call(
        _k2_body,
        grid=(N_ // BN,),
        in_specs=[_rb(BN, F_)] + [_full((F_, F_)), _full((1, F_))] * 3,
        out_specs=[_rb(BN, F_), _rb(BN, 2 * F_)],
        out_shape=[jax.ShapeDtypeStruct((N_, F_), jnp.float32),
                   jax.ShapeDtypeStruct((N_, 2 * F_), jnp.float32)],
    )(h, c['Wq'], c['bq'].reshape(1, F_), c['Wk'], c['bk'].reshape(1, F_),
      c['Wv'], c['bv'].reshape(1, F_))


def _edge_math(qg, kvg, ek, ev, gate, sbfl):
    return pl.pallas_call(
        _k4_body,
        grid=(E_ // BE,),
        in_specs=[_rb(BE, F_), _rb(BE, 2 * F_)] + [_rb(BE, F_)] * 3
                 + [_rb(BE, H_)],
        out_specs=[_rb(BE, F_), _rb(BE, F_)],
        out_shape=[jax.ShapeDtypeStruct((E_, F_), jnp.float32),
                   jax.ShapeDtypeStruct((E_, F_), jnp.float32)],
    )(qg, kvg, ek, ev, gate, sbfl)


def _post(hsum2, res0, batch_col, p, i):
    (Wb1, bb1), (Wb2, bb2) = p['bf_skip'][i]
    Wd, bd = p['dense_bf'][i]
    ((Wa1, ba1), (Wa2, ba2)), ((Wa3, ba3), (Wa4, ba4)) = p['af_skip'][i]
    grid = N_ // BN
    h, S1, S2, cnt = pl.pallas_call(
        _k6a_body,
        grid=(grid,),
        in_specs=[
            pl.BlockSpec((2, BN, F_), lambda i: (0, i, 0)),
            pl.BlockSpec((BN, 1), lambda i: (i, 0)),
        ],
        out_specs=[
            _rb(BN, F_),
            pl.BlockSpec((G_, F_), lambda i: (0, 0)),
            pl.BlockSpec((G_, F_), lambda i: (0, 0)),
            pl.BlockSpec((G_, 1), lambda i: (0, 0)),
        ],
        out_shape=[
            jax.ShapeDtypeStruct((N_, F_), jnp.float32),
            jax.ShapeDtypeStruct((G_, F_), jnp.float32),
            jax.ShapeDtypeStruct((G_, F_), jnp.float32),
            jax.ShapeDtypeStruct((G_, 1), jnp.float32),
        ],
    )(hsum2, batch_col)
    return pl.pallas_call(
        _k6b_body,
        grid=(grid,),
        in_specs=[
            _rb(BN, F_),
            pl.BlockSpec((BN, 1), lambda i: (i, 0)),
            pl.BlockSpec((G_, F_), lambda i: (0, 0)),
            pl.BlockSpec((G_, F_), lambda i: (0, 0)),
            pl.BlockSpec((G_, 1), lambda i: (0, 0)),
            _rb(BN, F_),
        ] + [_full((F_, F_)), _full((1, F_))] * 7,
        out_specs=_rb(BN, F_),
        out_shape=jax.ShapeDtypeStruct((N_, F_), jnp.float32),
    )(h, batch_col, S1, S2, cnt, res0,
      Wb1, bb1.reshape(1, F_), Wb2, bb2.reshape(1, F_),
      Wd, bd.reshape(1, F_),
      Wa1, ba1.reshape(1, F_), Wa2, ba2.reshape(1, F_),
      Wa3, ba3.reshape(1, F_), Wa4, ba4.reshape(1, F_))


def _readout(out, batch_col, icl_col, p):
    (Wr1, br1), (Wr2, br2) = p['readout']
    return pl.pallas_call(
        _k7_body,
        out_shape=jax.ShapeDtypeStruct((G_, 1), jnp.float32),
    )(out, batch_col, icl_col, Wr1, br1.reshape(1, F_),
      Wr2, br2.reshape(1, 1))


def kernel(x, edge_index, edge_attr, edge_sbf, node_rbf, batch, is_cleave,
           num_graphs, params):
    src = edge_index[0]
    dst = edge_index[1]
    batch_col = batch.reshape(N_, 1)
    icl_col = is_cleave.reshape(N_, 1)
    zrows = jnp.zeros((_ZROWS, F_), jnp.float32)

    q0, kv0 = _qkv(x, params['convs'][0])
    g0 = _gather_qkv(q0, kv0, src, dst)
    ek, ev, gate, sbfl = _edge_precompute(edge_attr, edge_sbf, node_rbf, params)

    out = x
    for i in range(3):
        c = params['convs'][i]
        if i == 0:
            qg, kvg = g0
        else:
            q, kv = _qkv(out, c)
            qg, kvg = _gather_qkv(q, kv, src, dst)
        a0, a1 = _edge_math(qg, kvg, ek[i], ev[i], gate[i], sbfl[i])
        hsum2 = _scatter_sum(a0, a1, dst, zrows)
        out = _post(hsum2, out, batch_col, params, i)

    res = _readout(out, batch_col, icl_col, params)
    return res.reshape(-1)


# speedup vs baseline: 28.3602x; 1.0706x over previous
import functools

import jax
import jax.numpy as jnp
from jax import lax
from jax.experimental import pallas as pl
from jax.experimental.pallas import tpu as pltpu
from jax.experimental.pallas import tpu_sc as plsc

N_ = 10000
E_ = 160000
F_ = 128
H_ = 8
CH_ = 16
G_ = 128
EPS = 1e-8

BE = 2000
BN = 2000

_NW = 32
_SC_CHUNK = 40
_SC_ITERS = E_ // (_SC_CHUNK * _NW)
_ZROWS = 80
_NZCH = N_ // _ZROWS
_ZITER = -(-_NZCH // 16)


def _silu(t):
    return t * jax.nn.sigmoid(t)


def _dot(a, b):
    return jnp.dot(a, b, precision=lax.Precision.HIGHEST)


def _dotw(a, b):
    return jnp.dot(a, b, precision=lax.Precision.DEFAULT)


def _dotc0(a, b):
    return lax.dot_general(a, b, (((0,), (0,)), ((), ())),
                           precision=lax.Precision.HIGHEST)


def _head_mats():
    hsel = (lax.broadcasted_iota(jnp.int32, (F_, H_), 0) // CH_
            == lax.broadcasted_iota(jnp.int32, (F_, H_), 1)).astype(jnp.float32)
    hexp = (lax.broadcasted_iota(jnp.int32, (H_, F_), 1) // CH_
            == lax.broadcasted_iota(jnp.int32, (H_, F_), 0)).astype(jnp.float32)
    return hsel, hexp


def _onehot_graph(batch_col):
    return (batch_col == lax.broadcasted_iota(jnp.int32, (N_, G_), 1)
            ).astype(jnp.float32)


def _k1_body(ea_r, sbf_r, W0r, b0r, W1r, b1r, Wsr, bsr, *outs):
    a = _dotw(_silu(_dotw(ea_r[...], W0r[...]) + b0r[...]), W1r[...]) + b1r[...]
    sbf = sbf_r[...]
    Ws, bs = Wsr[...], bsr[...]
    outs[0][...] = a
    for i in range(3):
        outs[1 + i][...] = _dotw(sbf, Ws[i]) + bs[i][None, :]


def _k2_body(x_r, Wq, bq, Wk, bk, Wv, bv, q_o, kv_o):
    xv = x_r[...]
    q_o[...] = _dotw(xv, Wq[...]) + bq[...]
    kv_o[...] = jnp.concatenate(
        [_dotw(xv, Wk[...]) + bk[...], _dotw(xv, Wv[...]) + bv[...]], axis=1)


def _k4_body(qg_r, kvg_r, ea_r, rbf_r, s_r,
             Wek, bek, Wev, bev, Wg, bg, a0_o, a1_o):
    hsel, hexp = _head_mats()
    qg = qg_r[...]
    kvg = kvg_r[...]
    a = ea_r[...]
    kk = kvg[:, :F_] + _dotw(a, Wek[...]) + bek[...]
    logits = _dot(qg * kk, hsel) * 0.25 + s_r[...]
    ehf = _dot(jnp.exp(logits), hexp)
    gate = _dotw(rbf_r[...], Wg[...]) + bg[...]
    m = ehf * (kvg[:, F_:] + _dotw(a, Wev[...]) + bev[...]) * gate
    a0_o[...] = jnp.concatenate([m[:, :64], ehf[:, :64]], axis=1)
    a1_o[...] = jnp.concatenate([m[:, 64:], ehf[:, 64:]], axis=1)


def _onehot_block(batch_col):
    return (batch_col == lax.broadcasted_iota(jnp.int32, (BN, G_), 1)
            ).astype(jnp.float32)


def _k6a_body(hs_r, batch_r, h_o, S1_o, S2_o, cnt_o):
    i = pl.program_id(0)
    hs = hs_r[...]
    h = jnp.concatenate(
        [hs[0, :, :64] / (hs[0, :, 64:] + 1e-16),
         hs[1, :, :64] / (hs[1, :, 64:] + 1e-16)], axis=1)
    h_o[...] = h
    oh = _onehot_block(batch_r[...])
    ones = jnp.full((BN, 1), 1.0, jnp.float32)

    @pl.when(i == 0)
    def _():
        S1_o[...] = jnp.zeros_like(S1_o)
        S2_o[...] = jnp.zeros_like(S2_o)
        cnt_o[...] = jnp.zeros_like(cnt_o)

    S1_o[...] += _dotc0(oh, h)
    S2_o[...] += _dotc0(oh, h * h)
    cnt_o[...] += _dotc0(oh, ones)


def _k6b_body(h_r, batch_r, S1_r, S2_r, cnt_r, res0_r,
              Wb1, bb1, Wb2, bb2, Wd, bd,
              Wa1, ba1, Wa2, ba2, Wa3, ba3, Wa4, ba4, out_o):
    n = cnt_r[...] * jnp.float32(F_) + 1e-12
    mean = jnp.sum(S1_r[...], axis=1, keepdims=True) / n
    var = jnp.sum(S2_r[...], axis=1, keepdims=True) / n - mean * mean
    sd = jnp.sqrt(var + EPS) + EPS
    oh = _onehot_block(batch_r[...])
    h = (h_r[...] - _dot(oh, mean)) / _dot(oh, sd)
    h = h + _silu(_dotw(_silu(_dotw(h, Wb1[...]) + bb1[...]), Wb2[...]) + bb2[...])
    h = _silu(_dotw(h, Wd[...]) + bd[...])
    h = h + res0_r[...]
    h = h + _silu(_dotw(_silu(_dotw(h, Wa1[...]) + ba1[...]), Wa2[...]) + ba2[...])
    h = h + _silu(_dotw(_silu(_dotw(h, Wa3[...]) + ba3[...]), Wa4[...]) + ba4[...])
    out_o[...] = h


def _k7_body(out_r, batch_r, icl_r, Wr1, br1, Wr2, br2, res_o):
    o = out_r[...]
    ne = _dotw(_silu(_dotw(o, Wr1[...]) + br1[...]), Wr2[...]) + br2[...]
    ne = ne * icl_r[...].astype(jnp.float32)
    oh = _onehot_graph(batch_r[...])
    res_o[...] = _dotc0(oh, ne)


def _rb(bs, feat):
    return pl.BlockSpec((bs, feat), lambda i: (i, 0))


def _full(shape):
    nd = len(shape)
    return pl.BlockSpec(shape, lambda i: (0,) * nd)


_NB = 5


def _gather_qkv(qT, kvT, src, dst):
    mesh = plsc.VectorSubcoreMesh(core_axis_name="c", subcore_axis_name="s")
    C = _SC_CHUNK
    nch = _SC_ITERS

    scratch = (
        [pltpu.VMEM((C,), jnp.int32)] * _NB
        + [pltpu.VMEM((C,), jnp.int32)] * _NB
        + [pltpu.VMEM((C, F_), jnp.float32)] * _NB
        + [pltpu.VMEM((C, 2 * F_), jnp.float32)] * _NB
        + [pltpu.SemaphoreType.DMA] * (2 * _NB)
    )

    @functools.partial(
        pl.kernel,
        out_type=(jax.ShapeDtypeStruct((E_, F_), jnp.float32),
                  jax.ShapeDtypeStruct((E_, 2 * F_), jnp.float32)),
        mesh=mesh,
        scratch_types=scratch,
    )
    def gk(qT_h, kvT_h, src_h, dst_h, qg_h, kvg_h, *scr):
        sidx = scr[0:_NB]
        didx = scr[_NB:2 * _NB]
        qb = scr[2 * _NB:3 * _NB]
        kvb = scr[3 * _NB:4 * _NB]
        gsem = scr[4 * _NB:5 * _NB]
        wsem = scr[5 * _NB:6 * _NB]
        wid = lax.axis_index("s") * 2 + lax.axis_index("c")

        def base(t):
            return (t * _NW + wid) * C

        def fire(t, b):
            pltpu.sync_copy(src_h.at[pl.ds(base(t), C)], sidx[b])
            pltpu.sync_copy(dst_h.at[pl.ds(base(t), C)], didx[b])
            pltpu.async_copy(qT_h.at[didx[b]], qb[b], gsem[b])
            pltpu.async_copy(kvT_h.at[sidx[b]], kvb[b], gsem[b])

        def drain_g(b):
            pltpu.make_async_copy(qT_h.at[didx[b]], qb[b], gsem[b]).wait()
            pltpu.make_async_copy(kvT_h.at[sidx[b]], kvb[b], gsem[b]).wait()

        def fire_wb(t, b):
            pltpu.async_copy(qb[b], qg_h.at[pl.ds(base(t), C)], wsem[b])
            pltpu.async_copy(kvb[b], kvg_h.at[pl.ds(base(t), C)], wsem[b])

        def drain_wb(t, b):
            pltpu.make_async_copy(qb[b], qg_h.at[pl.ds(base(t), C)], wsem[b]).wait()
            pltpu.make_async_copy(kvb[b], kvg_h.at[pl.ds(base(t), C)], wsem[b]).wait()

        fire(0, 0)
        fire(1, 1)

        def step(s, carry):
            for b in range(_NB):
                t = s * _NB + b
                tb = (b + 2) % _NB

                @pl.when(t >= 3)
                def _():
                    drain_wb(t - 3, tb)

                @pl.when(t + 2 < nch)
                def _():
                    fire(t + 2, tb)

                drain_g(b)
                fire_wb(t, b)
            return carry

        lax.fori_loop(0, nch // _NB, step, 0)
        drain_wb(nch - 3, (nch - 3) % _NB)
        drain_wb(nch - 2, (nch - 2) % _NB)
        drain_wb(nch - 1, (nch - 1) % _NB)

    return gk(qT, kvT, src, dst)


def _scatter_sum(a0, a1, dst, zrows):
    mesh = plsc.VectorSubcoreMesh(core_axis_name="c", subcore_axis_name="s")
    n_iter = E_ // (_SC_CHUNK * 16)

    C = _SC_CHUNK
    scratch = (
        [pltpu.VMEM((C,), jnp.int32)] * _NB
        + [pltpu.VMEM((C, F_), jnp.float32)] * _NB
        + [pltpu.SemaphoreType.DMA] * (2 * _NB)
        + [
            pltpu.VMEM((_ZROWS, F_), jnp.float32),
            pltpu.VMEM_SHARED((N_, F_), jnp.float32),
        ]
    )

    @functools.partial(
        pl.kernel,
        out_type=jax.ShapeDtypeStruct((2, N_, F_), jnp.float32),
        mesh=mesh,
        scratch_types=scratch,
    )
    def sk(a0_h, a1_h, dst_h, zrows_h, hsum_h, *scr):
        didx = scr[0:_NB]
        mb = scr[_NB:2 * _NB]
        rsem = scr[2 * _NB:3 * _NB]
        ssem = scr[3 * _NB:4 * _NB]
        rbuf, hs_sh = scr[4 * _NB], scr[4 * _NB + 1]
        cid = lax.axis_index("c")
        sid = lax.axis_index("s")
        pltpu.sync_copy(zrows_h, rbuf)

        def zbody(j, carry):
            zc = j * 16 + sid

            @pl.when(zc < _NZCH)
            def _():
                r0 = zc * _ZROWS
                pltpu.sync_copy(rbuf, hs_sh.at[pl.ds(r0, _ZROWS)])

            return carry

        lax.fori_loop(0, _ZITER, zbody, 0)
        plsc.subcore_barrier()

        def base(t):
            return (t * 16 + sid) * C

        def fire_r(t, b):
            pltpu.async_copy(dst_h.at[pl.ds(base(t), C)], didx[b], rsem[b])

            @pl.when(cid == 0)
            def _():
                pltpu.async_copy(a0_h.at[pl.ds(base(t), C)], mb[b], rsem[b])

            @pl.when(cid == 1)
            def _():
                pltpu.async_copy(a1_h.at[pl.ds(base(t), C)], mb[b], rsem[b])

        def drain_r(t, b):
            pltpu.make_async_copy(dst_h.at[pl.ds(base(t), C)], didx[b],
                                  rsem[b]).wait()
            pltpu.make_async_copy(a0_h.at[pl.ds(base(t), C)], mb[b],
                                  rsem[b]).wait()

        def fire_s(b):
            pltpu.async_copy(mb[b], hs_sh.at[didx[b]], ssem[b], add=True)

        def drain_s(b):
            pltpu.make_async_copy(mb[b], hs_sh.at[didx[b]], ssem[b]).wait()

        fire_r(0, 0)
        fire_r(1, 1)

        def body(s, carry):
            for b in range(_NB):
                t = s * _NB + b
                tb = (b + 2) % _NB

                @pl.when(t >= 3)
                def _():
                    drain_s(tb)

                @pl.when(t + 2 < n_iter)
                def _():
                    fire_r(t + 2, tb)

                drain_r(t, b)
                fire_s(b)
            return carry

        lax.fori_loop(0, n_iter // _NB, body, 0)
        drain_s((n_iter - 3) % _NB)
        drain_s((n_iter - 2) % _NB)
        drain_s((n_iter - 1) % _NB)
        plsc.subcore_barrier()

        def fbody(j, carry):
            zc = j * 16 + sid

            @pl.when(zc < _NZCH)
            def _():
                r0 = zc * _ZROWS
                pltpu.sync_copy(hs_sh.at[pl.ds(r0, _ZROWS)], rbuf)
                pltpu.sync_copy(rbuf, hsum_h.at[cid, pl.ds(r0, _ZROWS)])

            return carry

        lax.fori_loop(0, _ZITER, fbody, 0)

    return sk(a0, a1, dst, zrows)


def _edge_precompute(edge_attr, edge_sbf, p):
    (W0, b0), (W1, b1) = p['edgenn']
    convs = p['convs']
    Ws = jnp.stack([c['Wsbf'] for c in convs])
    bs = jnp.stack([c['bsbf'] for c in convs])
    grid = E_ // BE
    outs = pl.pallas_call(
        _k1_body,
        grid=(grid,),
        in_specs=[
            _rb(BE, F_), _rb(BE, 112),
            _full((F_, F_)), _full((1, F_)), _full((F_, F_)), _full((1, F_)),
            _full((3, 112, H_)), _full((3, H_)),
        ],
        out_specs=[_rb(BE, F_)] + [_rb(BE, H_)] * 3,
        out_shape=[jax.ShapeDtypeStruct((E_, F_), jnp.float32)]
        + [jax.ShapeDtypeStruct((E_, H_), jnp.float32)] * 3,
    )(edge_attr, edge_sbf,
      W0, b0.reshape(1, F_), W1, b1.reshape(1, F_), Ws, bs)
    return outs[0], outs[1:4]


def _qkv(h, c):
    return pl.pallas_call(
        _k2_body,
        grid=(N_ // BN,),
        in_specs=[_rb(BN, F_)] + [_full((F_, F_)), _full((1, F_))] * 3,
        out_specs=[_rb(BN, F_), _rb(BN, 2 * F_)],
        out_shape=[jax.ShapeDtypeStruct((N_, F_), jnp.float32),
                   jax.ShapeDtypeStruct((N_, 2 * F_), jnp.float32)],
    )(h, c['Wq'], c['bq'].reshape(1, F_), c['Wk'], c['bk'].reshape(1, F_),
      c['Wv'], c['bv'].reshape(1, F_))


def _edge_math(qg, kvg, ea, node_rbf, sbfl, c):
    return pl.pallas_call(
        _k4_body,
        grid=(E_ // BE,),
        in_specs=[_rb(BE, F_), _rb(BE, 2 * F_), _rb(BE, F_), _rb(BE, 16),
                  _rb(BE, H_),
                  _full((F_, F_)), _full((1, F_)),
                  _full((F_, F_)), _full((1, F_)),
                  _full((16, F_)), _full((1, F_))],
        out_specs=[_rb(BE, F_), _rb(BE, F_)],
        out_shape=[jax.ShapeDtypeStruct((E_, F_), jnp.float32),
                   jax.ShapeDtypeStruct((E_, F_), jnp.float32)],
    )(qg, kvg, ea, node_rbf, sbfl,
      c['We_k'], c['be_k'].reshape(1, F_),
      c['We_v'], c['be_v'].reshape(1, F_),
      c['Wrbf'], c['brbf'].reshape(1, F_))


def _post(hsum2, res0, batch_col, p, i):
    (Wb1, bb1), (Wb2, bb2) = p['bf_skip'][i]
    Wd, bd = p['dense_bf'][i]
    ((Wa1, ba1), (Wa2, ba2)), ((Wa3, ba3), (Wa4, ba4)) = p['af_skip'][i]
    grid = N_ // BN
    h, S1, S2, cnt = pl.pallas_call(
        _k6a_body,
        grid=(grid,),
        in_specs=[
            pl.BlockSpec((2, BN, F_), lambda i: (0, i, 0)),
            pl.BlockSpec((BN, 1), lambda i: (i, 0)),
        ],
        out_specs=[
            _rb(BN, F_),
            pl.BlockSpec((G_, F_), lambda i: (0, 0)),
            pl.BlockSpec((G_, F_), lambda i: (0, 0)),
            pl.BlockSpec((G_, 1), lambda i: (0, 0)),
        ],
        out_shape=[
            jax.ShapeDtypeStruct((N_, F_), jnp.float32),
            jax.ShapeDtypeStruct((G_, F_), jnp.float32),
            jax.ShapeDtypeStruct((G_, F_), jnp.float32),
            jax.ShapeDtypeStruct((G_, 1), jnp.float32),
        ],
    )(hsum2, batch_col)
    return pl.pallas_call(
        _k6b_body,
        grid=(grid,),
        in_specs=[
            _rb(BN, F_),
            pl.BlockSpec((BN, 1), lambda i: (i, 0)),
            pl.BlockSpec((G_, F_), lambda i: (0, 0)),
            pl.BlockSpec((G_, F_), lambda i: (0, 0)),
            pl.BlockSpec((G_, 1), lambda i: (0, 0)),
            _rb(BN, F_),
        ] + [_full((F_, F_)), _full((1, F_))] * 7,
        out_specs=_rb(BN, F_),
        out_shape=jax.ShapeDtypeStruct((N_, F_), jnp.float32),
    )(h, batch_col, S1, S2, cnt, res0,
      Wb1, bb1.reshape(1, F_), Wb2, bb2.reshape(1, F_),
      Wd, bd.reshape(1, F_),
      Wa1, ba1.reshape(1, F_), Wa2, ba2.reshape(1, F_),
      Wa3, ba3.reshape(1, F_), Wa4, ba4.reshape(1, F_))


def _readout(out, batch_col, icl_col, p):
    (Wr1, br1), (Wr2, br2) = p['readout']
    return pl.pallas_call(
        _k7_body,
        out_shape=jax.ShapeDtypeStruct((G_, 1), jnp.float32),
    )(out, batch_col, icl_col, Wr1, br1.reshape(1, F_),
      Wr2, br2.reshape(1, 1))


def kernel(x, edge_index, edge_attr, edge_sbf, node_rbf, batch, is_cleave,
           num_graphs, params):
    src = edge_index[0]
    dst = edge_index[1]
    batch_col = batch.reshape(N_, 1)
    icl_col = is_cleave.reshape(N_, 1)
    zrows = jnp.zeros((_ZROWS, F_), jnp.float32)

    q0, kv0 = _qkv(x, params['convs'][0])
    g0 = _gather_qkv(q0, kv0, src, dst)
    ea, sbfl = _edge_precompute(edge_attr, edge_sbf, params)

    out = x
    for i in range(3):
        c = params['convs'][i]
        if i == 0:
            qg, kvg = g0
        else:
            q, kv = _qkv(out, c)
            qg, kvg = _gather_qkv(q, kv, src, dst)
        a0, a1 = _edge_math(qg, kvg, ea, node_rbf, sbfl[i], c)
        hsum2 = _scatter_sum(a0, a1, dst, zrows)
        out = _post(hsum2, out, batch_col, params, i)

    res = _readout(out, batch_col, icl_col, params)
    return res.reshape(-1)
